# Initial kernel scaffold; baseline (speedup 1.0000x reference)
import jax
import jax.numpy as jnp
from jax.experimental import pallas as pl


def kernel(in_feat, edge_index, l1_W, l1_b, l2_W, l2_b, c1_W0, c1_W1, c1_b, c2_W0, c2_W1, c2_b, l3_W, l3_b, l4_W, l4_b):
    raise NotImplementedError("write your pallas kernel here")


# speedup vs baseline: 10.7961x; 10.7961x over previous
import functools

import jax
import jax.numpy as jnp
from jax import lax
from jax.experimental import pallas as pl
from jax.experimental.pallas import tpu as pltpu
from jax.experimental.pallas import tpu_sc as plsc

N = 10000
D = 128
E = 320000
OUT_C = 2

NC = 2
NS = 16
NW = NC * NS
EPT = E // NW
BLK = 128
NBLK = -(-EPT // BLK)
EPT_PAD = NBLK * BLK
NPAD = N + 16
RZ = NPAD // NS
RO = 624
RO_TAIL = N - NS * RO
DEGW = 16

ROWS = 1000
GRID = N // ROWS


def _sc_mesh():
    return plsc.VectorSubcoreMesh(core_axis_name="c", subcore_axis_name="s")


@functools.partial(
    pl.kernel,
    out_type=jax.ShapeDtypeStruct((NC, N, D), jnp.float32),
    mesh=_sc_mesh(),
    scratch_types=[
        pltpu.VMEM((NBLK, BLK), jnp.int32),
        pltpu.VMEM((BLK, D), jnp.float32),
        pltpu.VMEM_SHARED((NPAD, D), jnp.float32),
    ],
)
def _deg_kernel(srch, deg_out, idx_v, buf, acc):
    c = lax.axis_index("c")
    s = lax.axis_index("s")
    wid = c * NS + s

    def _fill(val):
        def row(i, carry):
            for k2 in range(D // 16):
                buf[i, pl.ds(k2 * 16, 16)] = jnp.full((16,), val, jnp.float32)
            return carry
        lax.fori_loop(0, BLK, row, 0)

    _fill(0.0)
    zbase = s * RZ
    for k in range(RZ // BLK):
        pltpu.sync_copy(buf, acc.at[pl.ds(zbase + k * BLK, BLK)])
    rem = RZ % BLK
    pltpu.sync_copy(buf.at[pl.ds(0, rem)],
                    acc.at[pl.ds(zbase + (RZ // BLK) * BLK, rem)])
    plsc.subcore_barrier()

    _fill(1.0)
    pltpu.sync_copy(srch.at[wid], idx_v)

    def blk(j, carry):
        pltpu.sync_copy(buf, acc.at[idx_v.at[j]], add=True)
        return carry
    lax.fori_loop(0, NBLK, blk, 0)
    plsc.subcore_barrier()

    pltpu.sync_copy(acc.at[pl.ds(s * RO, RO)],
                    deg_out.at[c, pl.ds(s * RO, RO)])

    @pl.when(s == 0)
    def _tail():
        pltpu.sync_copy(acc.at[pl.ds(NS * RO, RO_TAIL)],
                        deg_out.at[c, pl.ds(NS * RO, RO_TAIL)])


@functools.partial(
    pl.kernel,
    out_type=jax.ShapeDtypeStruct((NC, N, D), jnp.float32),
    mesh=_sc_mesh(),
    scratch_types=[
        pltpu.VMEM((NBLK, BLK), jnp.int32),
        pltpu.VMEM((NBLK, BLK), jnp.int32),
        pltpu.VMEM((BLK, D), jnp.float32),
        pltpu.VMEM_SHARED((NPAD, D), jnp.float32),
        pltpu.SemaphoreType.DMA,
    ],
)
def _prop_kernel(q, srcg, dstb, s_out, src_v, dst_v, buf, acc, sem):
    c = lax.axis_index("c")
    s = lax.axis_index("s")
    wid = c * NS + s

    def zrow(i, carry):
        for k2 in range(D // 16):
            buf[i, pl.ds(k2 * 16, 16)] = jnp.zeros((16,), jnp.float32)
        return carry
    lax.fori_loop(0, BLK, zrow, 0)
    zbase = s * RZ
    for k in range(RZ // BLK):
        pltpu.sync_copy(buf, acc.at[pl.ds(zbase + k * BLK, BLK)])
    rem = RZ % BLK
    pltpu.sync_copy(buf.at[pl.ds(0, rem)],
                    acc.at[pl.ds(zbase + (RZ // BLK) * BLK, rem)])
    plsc.subcore_barrier()

    pltpu.sync_copy(srcg.at[wid], src_v)
    pltpu.sync_copy(dstb.at[wid], dst_v)

    def blk(j, carry):
        pltpu.async_copy(q.at[src_v.at[j]], buf, sem).wait()
        pltpu.sync_copy(buf, acc.at[dst_v.at[j]], add=True)
        return carry
    lax.fori_loop(0, NBLK, blk, 0)
    plsc.subcore_barrier()

    pltpu.sync_copy(acc.at[pl.ds(s * RO, RO)],
                    s_out.at[c, pl.ds(s * RO, RO)])

    @pl.when(s == 0)
    def _tail():
        pltpu.sync_copy(acc.at[pl.ds(NS * RO, RO_TAIL)],
                        s_out.at[c, pl.ds(NS * RO, RO_TAIL)])


def _dot(a, b):
    return jnp.dot(a, b, preferred_element_type=jnp.float32)


def _tc1_body(x_ref, d0_ref, d1_ref, w1_ref, b1_ref, w2_ref, b2_ref, cw1_ref,
              h_ref, q_ref, dis_ref):
    h = jax.nn.relu(_dot(x_ref[...], w1_ref[...]) + b1_ref[...])
    h = jax.nn.relu(_dot(h, w2_ref[...]) + b2_ref[...])
    deg = d0_ref[:, 0:1] + d1_ref[:, 0:1]
    dis = jnp.where(deg > 0, 1.0 / jnp.sqrt(jnp.maximum(deg, 1e-12)), 0.0)
    h_ref[...] = h
    q_ref[...] = -dis * _dot(h, cw1_ref[...])
    dis_ref[...] = jnp.broadcast_to(dis, (ROWS, DEGW))


def _tc2_body(h_ref, s0_ref, s1_ref, dis_ref, w0_ref, b_ref, cwn_ref,
              h2_ref, q2_ref):
    dis = dis_ref[:, 0:1]
    h2 = _dot(h_ref[...], w0_ref[...]) + dis * (s0_ref[...] + s1_ref[...]) + b_ref[...]
    h2_ref[...] = h2
    q2_ref[...] = -dis * _dot(h2, cwn_ref[...])


def _tc3_body(h_ref, s0_ref, s1_ref, dis_ref, w0_ref, b_ref, w3_ref, b3_ref,
              w4_ref, b4_ref, o_ref):
    dis = dis_ref[:, 0:1]
    h3 = _dot(h_ref[...], w0_ref[...]) + dis * (s0_ref[...] + s1_ref[...]) + b_ref[...]
    h4 = jax.nn.relu(_dot(h3, w3_ref[...]) + b3_ref[...])
    o_ref[...] = _dot(h4, w4_ref[...]) + b4_ref[...]


def _row_spec(w):
    return pl.BlockSpec((ROWS, w), lambda i: (i, 0))


def _full_spec(r, w):
    return pl.BlockSpec((r, w), lambda i: (0, 0))


def _tc1(x, d0, d1, w1, b1, w2, b2, cw1):
    return pl.pallas_call(
        _tc1_body,
        grid=(GRID,),
        in_specs=[
            _row_spec(D), _row_spec(D), _row_spec(D),
            _full_spec(D, D), _full_spec(1, D),
            _full_spec(D, D), _full_spec(1, D),
            _full_spec(D, D),
        ],
        out_specs=[_row_spec(D), _row_spec(D), _row_spec(DEGW)],
        out_shape=[
            jax.ShapeDtypeStruct((N, D), jnp.float32),
            jax.ShapeDtypeStruct((N, D), jnp.float32),
            jax.ShapeDtypeStruct((N, DEGW), jnp.float32),
        ],
    )(x, d0, d1, w1, b1, w2, b2, cw1)


def _tc2(h, s0, s1, dis, w0, b, cwn):
    return pl.pallas_call(
        _tc2_body,
        grid=(GRID,),
        in_specs=[
            _row_spec(D), _row_spec(D), _row_spec(D), _row_spec(DEGW),
            _full_spec(D, D), _full_spec(1, D), _full_spec(D, D),
        ],
        out_specs=[_row_spec(D), _row_spec(D)],
        out_shape=[
            jax.ShapeDtypeStruct((N, D), jnp.float32),
            jax.ShapeDtypeStruct((N, D), jnp.float32),
        ],
    )(h, s0, s1, dis, w0, b, cwn)


def _tc3(h, s0, s1, dis, w0, b, w3, b3, w4, b4):
    return pl.pallas_call(
        _tc3_body,
        grid=(GRID,),
        in_specs=[
            _row_spec(D), _row_spec(D), _row_spec(D), _row_spec(DEGW),
            _full_spec(D, D), _full_spec(1, D),
            _full_spec(D, D), _full_spec(1, D),
            _full_spec(D, OUT_C), _full_spec(1, OUT_C),
        ],
        out_specs=_row_spec(OUT_C),
        out_shape=jax.ShapeDtypeStruct((N, OUT_C), jnp.float32),
    )(h, s0, s1, dis, w0, b, w3, b3, w4, b4)


def kernel(in_feat, edge_index, l1_W, l1_b, l2_W, l2_b,
           c1_W0, c1_W1, c1_b, c2_W0, c2_W1, c2_b,
           l3_W, l3_b, l4_W, l4_b):
    src = edge_index[0].reshape(NW, EPT)
    dst = edge_index[1].reshape(NW, EPT)
    pad = EPT_PAD - EPT
    srcg = jnp.pad(src, ((0, 0), (0, pad))).reshape(NW, NBLK, BLK)
    srch = jnp.pad(src, ((0, 0), (0, pad)), constant_values=N).reshape(NW, NBLK, BLK)
    dstb = jnp.pad(dst, ((0, 0), (0, pad)), constant_values=N).reshape(NW, NBLK, BLK)

    b1 = l1_b.reshape(1, D)
    b2 = l2_b.reshape(1, D)
    cb1 = c1_b.reshape(1, D)
    cb2 = c2_b.reshape(1, D)
    b3 = l3_b.reshape(1, D)
    b4 = l4_b.reshape(1, OUT_C)

    deg2 = _deg_kernel(srch)
    h1, q1, dis16 = _tc1(in_feat, deg2[0], deg2[1], l1_W, b1, l2_W, b2, c1_W1)
    s1 = _prop_kernel(q1, srcg, dstb)
    h2, q2 = _tc2(h1, s1[0], s1[1], dis16, c1_W0, cb1, c2_W1)
    s2 = _prop_kernel(q2, srcg, dstb)
    return _tc3(h2, s2[0], s2[1], dis16, c2_W0, cb2, l3_W, b3, l4_W, b4)
